# Initial kernel scaffold; baseline (speedup 1.0000x reference)
import jax
import jax.numpy as jnp
from jax.experimental import pallas as pl


def kernel(X_Node, X_Neis, dg_list, node_features, W_xi, b_xi, W_rou, b_rou, W_lin, b_lin):
    raise NotImplementedError("write your pallas kernel here")


# speedup vs baseline: 2.1420x; 2.1420x over previous
import functools

import jax
import jax.numpy as jnp
from jax import lax
from jax.experimental import pallas as pl
from jax.experimental.pallas import tpu as pltpu
from jax.experimental.pallas import tpu_sc as plsc

V = 2048
LN = 128
S = 32
MU = 0.9
E = 32768

NC = 2
NS = 16
NW = NC * NS
EPW = E // NW
CH = 128
NCH = EPW // CH

_mesh = plsc.VectorSubcoreMesh(core_axis_name="c", subcore_axis_name="s")

_DEF = lax.Precision.DEFAULT
_HI = lax.Precision.HIGHEST


@functools.partial(
    pl.kernel,
    out_type=jax.ShapeDtypeStruct((NW, V * 16), jnp.float32),
    mesh=_mesh,
    scratch_types=[
        pltpu.VMEM((EPW,), jnp.int32),
        pltpu.VMEM((V * 16,), jnp.float32),
    ],
)
def _hist(xn_hbm, zeros_hbm, deg_hbm, idx_v, deg_v):
    cid = lax.axis_index("c")
    sid = lax.axis_index("s")
    wid = sid * NC + cid
    base = wid * EPW
    pltpu.sync_copy(zeros_hbm, deg_v)
    pltpu.sync_copy(xn_hbm.at[pl.ds(base, EPW)], idx_v)

    def group(g, carry):
        uvec = idx_v[pl.ds(g * 16, 16)]
        for l in range(16):
            u = uvec[l]
            dsl = pl.ds(u * 16, 16)
            deg_v[dsl] = deg_v[dsl] + 1.0
        return carry

    lax.fori_loop(0, EPW // 16, group, 0)
    pltpu.sync_copy(deg_v, deg_hbm.at[wid])


def _make_reduce(width, bk):
    def body(in_ref, out_ref):
        out_ref[...] = jnp.sum(in_ref[...], axis=0, keepdims=True)[None]

    return pl.pallas_call(
        body,
        grid=(width // bk,),
        in_specs=[pl.BlockSpec((NW, bk), lambda i: (0, i))],
        out_specs=pl.BlockSpec((1, 1, bk), lambda i: (i, 0, 0)),
        out_shape=jax.ShapeDtypeStruct((width // bk, 1, bk), jnp.float32),
    )


_reduce_part = _make_reduce(V * S, 8192)
_reduce_deg = _make_reduce(V * 16, 8192)


def _prep_body(nf_ref, wr_ref, br_ref, deg16_ref, p_ref, out_ref):
    z = jnp.dot(nf_ref[...], wr_ref[...],
                preferred_element_type=jnp.float32, precision=_DEF)
    out_ref[...] = (jnp.tanh(z + br_ref[...])
                    + jnp.dot(deg16_ref[...], p_ref[...],
                              preferred_element_type=jnp.float32,
                              precision=_HI))


_prep = pl.pallas_call(
    _prep_body,
    out_shape=jax.ShapeDtypeStruct((V, LN), jnp.float32),
)


@functools.partial(
    pl.kernel,
    out_type=(
        jax.ShapeDtypeStruct((E, 2 * LN), jnp.float32),
        jax.ShapeDtypeStruct((E, LN), jnp.float32),
    ),
    mesh=_mesh,
    scratch_types=[
        pltpu.VMEM((CH,), jnp.int32),
        pltpu.VMEM((CH,), jnp.int32),
        pltpu.VMEM((CH, LN), jnp.float32),
        pltpu.VMEM((CH, LN), jnp.float32),
        pltpu.VMEM((CH, LN), jnp.float32),
        pltpu.SemaphoreType.DMA,
    ],
)
def _gather(xn_hbm, xw_hbm, nf_hbm, bd_hbm,
            xcat_hbm, bdg_hbm,
            idxn_v, idxw_v, bufn_v, bufw_v, bufb_v, sem):
    cid = lax.axis_index("c")
    sid = lax.axis_index("s")
    wid = sid * NC + cid
    base = wid * EPW
    for k in range(NCH):
        off = base + k * CH
        pltpu.sync_copy(xn_hbm.at[pl.ds(off, CH)], idxn_v)
        pltpu.sync_copy(xw_hbm.at[pl.ds(off, CH)], idxw_v)
        cp1 = pltpu.async_copy(nf_hbm.at[idxn_v], bufn_v, sem)
        cp2 = pltpu.async_copy(nf_hbm.at[idxw_v], bufw_v, sem)
        cp3 = pltpu.async_copy(bd_hbm.at[idxn_v], bufb_v, sem)
        cp1.wait()
        cp2.wait()
        cp3.wait()
        pltpu.sync_copy(bufn_v, xcat_hbm.at[pl.ds(off, CH), pl.ds(0, LN)])
        pltpu.sync_copy(bufw_v, xcat_hbm.at[pl.ds(off, CH), pl.ds(LN, LN)])
        pltpu.sync_copy(bufb_v, bdg_hbm.at[pl.ds(off, CH)])


BE = 2048


def _dense_body(xcat_ref, bdg_ref, dg_ref, w1_ref, bxi_ref,
                tm_ref, g_ref, out_ref):
    z = jnp.dot(xcat_ref[...], w1_ref[...],
                preferred_element_type=jnp.float32, precision=_DEF) + bxi_ref[...]
    a = jnp.tanh(z)
    hgt = jnp.dot(bdg_ref[...], tm_ref[...],
                  preferred_element_type=jnp.float32, precision=_DEF)
    core = jnp.dot(a * hgt, g_ref[...],
                   preferred_element_type=jnp.float32, precision=_HI)
    deg = bdg_ref[...][:, S:S + 1]
    hn = core * deg * ((MU / S) / dg_ref[...]) + bdg_ref[...][:, :S]
    out_ref[...] = hn.astype(jnp.bfloat16).astype(jnp.float32)


_dense = pl.pallas_call(
    _dense_body,
    grid=(E // BE,),
    in_specs=[
        pl.BlockSpec((BE, 2 * LN), lambda i: (i, 0)),
        pl.BlockSpec((BE, LN), lambda i: (i, 0)),
        pl.BlockSpec((BE, 1), lambda i: (i, 0)),
        pl.BlockSpec((2 * LN, S * S), lambda i: (0, 0)),
        pl.BlockSpec((1, S * S), lambda i: (0, 0)),
        pl.BlockSpec((LN, S * S), lambda i: (0, 0)),
        pl.BlockSpec((S * S, S), lambda i: (0, 0)),
    ],
    out_specs=pl.BlockSpec((BE, S), lambda i: (i, 0)),
    out_shape=jax.ShapeDtypeStruct((E, S), jnp.float32),
)


@functools.partial(
    pl.kernel,
    out_type=jax.ShapeDtypeStruct((NW, V * S), jnp.float32),
    mesh=_mesh,
    scratch_types=[
        pltpu.VMEM((EPW,), jnp.int32),
        pltpu.VMEM((CH * S,), jnp.float32),
        pltpu.VMEM((V * S,), jnp.float32),
    ],
)
def _scatter(xn_hbm, hnf_hbm, zeros_hbm, out_hbm, idx_v, rows_v, acc_v):
    cid = lax.axis_index("c")
    sid = lax.axis_index("s")
    wid = sid * NC + cid
    base = wid * EPW
    pltpu.sync_copy(zeros_hbm, acc_v)
    pltpu.sync_copy(xn_hbm.at[pl.ds(base, EPW)], idx_v)
    for k in range(NCH):
        pltpu.sync_copy(hnf_hbm.at[pl.ds((base + k * CH) * S, CH * S)], rows_v)

        def group(g, carry, k=k):
            uvec = idx_v[pl.ds(k * CH + g * 16, 16)]
            for l in range(16):
                u = uvec[l]
                e = g * 16 + l
                for h in range(S // 16):
                    sl = pl.ds(u * S + h * 16, 16)
                    acc_v[sl] = acc_v[sl] + rows_v[pl.ds(e * S + h * 16, 16)]
            return carry

        lax.fori_loop(0, CH // 16, group, 0)
    pltpu.sync_copy(acc_v, out_hbm.at[wid])


def _readout_body(nf_ref, ns2_ref, wl1_ref, wl2_ref, bl_ref, out_ref):
    logits = (jnp.dot(nf_ref[...], wl1_ref[...],
                      preferred_element_type=jnp.float32, precision=_DEF)
              + jnp.dot(ns2_ref[...], wl2_ref[...],
                        preferred_element_type=jnp.float32, precision=_DEF)
              + bl_ref[...])
    m = jnp.max(logits, axis=0, keepdims=True)
    e = jnp.exp(logits - m)
    out_ref[...] = e / jnp.sum(e, axis=0, keepdims=True)


_readout = pl.pallas_call(
    _readout_body,
    out_shape=jax.ShapeDtypeStruct((V, 3), jnp.float32),
)


def kernel(X_Node, X_Neis, dg_list, node_features, W_xi, b_xi,
           W_rou, b_rou, W_lin, b_lin):
    xn = X_Node.astype(jnp.int32)
    xw = X_Neis.astype(jnp.int32)
    nf = node_features

    degp = _hist(xn, jnp.zeros((V * 16,), jnp.float32))
    deg16 = _reduce_deg(degp).reshape(V, 16)

    wr_pad = jnp.zeros((LN, LN), jnp.float32).at[:, :S].set(W_rou.T)
    br_pad = jnp.zeros((1, LN), jnp.float32).at[:, :S].set(b_rou)
    P = jnp.zeros((16, LN), jnp.float32).at[0, S].set(1.0)
    BD = _prep(nf, wr_pad, br_pad, deg16, P)

    Xcat, BDg = _gather(xn, xw, nf, BD)

    j = jnp.arange(S * S)
    r = jnp.arange(S)
    Tm = (j[None, :] % S == r[:, None]).astype(jnp.float32)
    Tm_pad = jnp.zeros((LN, S * S), jnp.float32).at[:S].set(Tm)
    G = (j[:, None] // S == r[None, :]).astype(jnp.float32)

    Hn = _dense(Xcat, BDg, dg_list.reshape(E, 1),
                W_xi.T, b_xi.reshape(1, S * S), Tm_pad, G)

    partials = _scatter(xn, Hn.reshape(E * S), jnp.zeros((V * S,), jnp.float32))
    ns2 = _reduce_part(partials).reshape(V, S)

    out = _readout(nf, ns2, W_lin[:, :LN].T, W_lin[:, LN:].T,
                   b_lin.reshape(1, 3))
    return out


# speedup vs baseline: 3.0699x; 1.4332x over previous
import functools

import jax
import jax.numpy as jnp
from jax import lax
from jax.experimental import pallas as pl
from jax.experimental.pallas import tpu as pltpu
from jax.experimental.pallas import tpu_sc as plsc

V = 2048
LN = 128
S = 32
MU = 0.9
E = 32768

NC = 2
NS = 16
NW = NC * NS
EPW = E // NW
CH = 128
NCH = EPW // CH

_mesh = plsc.VectorSubcoreMesh(core_axis_name="c", subcore_axis_name="s")

_DEF = lax.Precision.DEFAULT
_HI = lax.Precision.HIGHEST


@functools.partial(
    pl.kernel,
    out_type=jax.ShapeDtypeStruct((NW, V * 16), jnp.float32),
    mesh=_mesh,
    scratch_types=[
        pltpu.VMEM((EPW,), jnp.int32),
        pltpu.VMEM((V * 16,), jnp.float32),
    ],
)
def _hist(xn_hbm, zeros_hbm, deg_hbm, idx_v, deg_v):
    cid = lax.axis_index("c")
    sid = lax.axis_index("s")
    wid = sid * NC + cid
    base = wid * EPW
    pltpu.sync_copy(zeros_hbm, deg_v)
    pltpu.sync_copy(xn_hbm.at[pl.ds(base, EPW)], idx_v)

    def group(g, carry):
        uvec = idx_v[pl.ds(g * 16, 16)]
        for l in range(16):
            u = uvec[l]
            dsl = pl.ds(u * 16, 16)
            deg_v[dsl] = deg_v[dsl] + 1.0
        return carry

    lax.fori_loop(0, EPW // 16, group, 0)
    pltpu.sync_copy(deg_v, deg_hbm.at[wid])


def _make_reduce(width, bk):
    def body(in_ref, out_ref):
        out_ref[...] = jnp.sum(in_ref[...], axis=0, keepdims=True)[None]

    return pl.pallas_call(
        body,
        grid=(width // bk,),
        in_specs=[pl.BlockSpec((NW, bk), lambda i: (0, i))],
        out_specs=pl.BlockSpec((1, 1, bk), lambda i: (i, 0, 0)),
        out_shape=jax.ShapeDtypeStruct((width // bk, 1, bk), jnp.float32),
    )


_reduce_part = _make_reduce(V * S, 8192)
_reduce_deg = _make_reduce(V * 16, 8192)


def _prep_body(nf_ref, wr_ref, br_ref, deg16_ref, p_ref, out_ref):
    z = jnp.dot(nf_ref[...], wr_ref[...],
                preferred_element_type=jnp.float32, precision=_DEF)
    out_ref[...] = (jnp.tanh(z + br_ref[...])
                    + jnp.dot(deg16_ref[...], p_ref[...],
                              preferred_element_type=jnp.float32,
                              precision=_HI))


_prep = pl.pallas_call(
    _prep_body,
    out_shape=jax.ShapeDtypeStruct((V, LN), jnp.float32),
)


@functools.partial(
    pl.kernel,
    out_type=(
        jax.ShapeDtypeStruct((E, 2 * LN), jnp.float32),
        jax.ShapeDtypeStruct((E, LN), jnp.float32),
    ),
    mesh=_mesh,
    scratch_types=[
        pltpu.VMEM((CH,), jnp.int32),
        pltpu.VMEM((CH,), jnp.int32),
        pltpu.VMEM((CH, LN), jnp.float32),
        pltpu.VMEM((CH, LN), jnp.float32),
        pltpu.VMEM((CH, LN), jnp.float32),
        pltpu.SemaphoreType.DMA,
    ],
)
def _gather(xn_hbm, xw_hbm, nf_hbm, bd_hbm,
            xcat_hbm, bdg_hbm,
            idxn_v, idxw_v, bufn_v, bufw_v, bufb_v, sem):
    cid = lax.axis_index("c")
    sid = lax.axis_index("s")
    wid = sid * NC + cid
    base = wid * EPW
    for k in range(NCH):
        off = base + k * CH
        pltpu.sync_copy(xn_hbm.at[pl.ds(off, CH)], idxn_v)
        pltpu.sync_copy(xw_hbm.at[pl.ds(off, CH)], idxw_v)
        cp1 = pltpu.async_copy(nf_hbm.at[idxn_v], bufn_v, sem)
        cp2 = pltpu.async_copy(nf_hbm.at[idxw_v], bufw_v, sem)
        cp3 = pltpu.async_copy(bd_hbm.at[idxn_v], bufb_v, sem)
        cp1.wait()
        cp2.wait()
        cp3.wait()
        pltpu.sync_copy(bufn_v, xcat_hbm.at[pl.ds(off, CH), pl.ds(0, LN)])
        pltpu.sync_copy(bufw_v, xcat_hbm.at[pl.ds(off, CH), pl.ds(LN, LN)])
        pltpu.sync_copy(bufb_v, bdg_hbm.at[pl.ds(off, CH)])


BE = 2048


def _dense_body(xcat_ref, bdg_ref, dg_ref, w1_ref, bxi_ref,
                tm_ref, f_ref, out_ref):
    z = jnp.dot(xcat_ref[...], w1_ref[...],
                preferred_element_type=jnp.float32, precision=_DEF) + bxi_ref[...]
    a = jnp.tanh(z)
    hgt = jnp.dot(bdg_ref[...], tm_ref[...],
                  preferred_element_type=jnp.float32, precision=_DEF)
    p = a * hgt
    acc = p[:, 0:128]
    for k in range(1, 8):
        acc = acc + p[:, 128 * k:128 * (k + 1)]
    core = jnp.dot(acc, f_ref[...],
                   preferred_element_type=jnp.float32, precision=_HI)
    deg = bdg_ref[...][:, S:S + 1]
    hn = core * deg * ((MU / S) / dg_ref[...]) + bdg_ref[...][:, :S]
    out_ref[...] = hn.astype(jnp.bfloat16).astype(jnp.float32)


_dense = pl.pallas_call(
    _dense_body,
    grid=(E // BE,),
    in_specs=[
        pl.BlockSpec((BE, 2 * LN), lambda i: (i, 0)),
        pl.BlockSpec((BE, LN), lambda i: (i, 0)),
        pl.BlockSpec((BE, 1), lambda i: (i, 0)),
        pl.BlockSpec((2 * LN, S * S), lambda i: (0, 0)),
        pl.BlockSpec((1, S * S), lambda i: (0, 0)),
        pl.BlockSpec((LN, S * S), lambda i: (0, 0)),
        pl.BlockSpec((LN, S), lambda i: (0, 0)),
    ],
    out_specs=pl.BlockSpec((BE, S), lambda i: (i, 0)),
    out_shape=jax.ShapeDtypeStruct((E, S), jnp.float32),
)


@functools.partial(
    pl.kernel,
    out_type=jax.ShapeDtypeStruct((NW, V * S), jnp.float32),
    mesh=_mesh,
    scratch_types=[
        pltpu.VMEM((EPW,), jnp.int32),
        pltpu.VMEM((CH * S,), jnp.float32),
        pltpu.VMEM((V * S,), jnp.float32),
    ],
)
def _scatter(xn_hbm, hnf_hbm, zeros_hbm, out_hbm, idx_v, rows_v, acc_v):
    cid = lax.axis_index("c")
    sid = lax.axis_index("s")
    wid = sid * NC + cid
    base = wid * EPW
    pltpu.sync_copy(zeros_hbm, acc_v)
    pltpu.sync_copy(xn_hbm.at[pl.ds(base, EPW)], idx_v)
    for k in range(NCH):
        pltpu.sync_copy(hnf_hbm.at[pl.ds((base + k * CH) * S, CH * S)], rows_v)

        def group(g, carry, k=k):
            uvec = idx_v[pl.ds(k * CH + g * 16, 16)]
            for l in range(16):
                u = uvec[l]
                e = g * 16 + l
                for h in range(S // 16):
                    sl = pl.ds(u * S + h * 16, 16)
                    acc_v[sl] = acc_v[sl] + rows_v[pl.ds(e * S + h * 16, 16)]
            return carry

        lax.fori_loop(0, CH // 16, group, 0)
    pltpu.sync_copy(acc_v, out_hbm.at[wid])


def _readout_body(nf_ref, ns2_ref, wl1_ref, wl2_ref, bl_ref, out_ref):
    logits = (jnp.dot(nf_ref[...], wl1_ref[...],
                      preferred_element_type=jnp.float32, precision=_DEF)
              + jnp.dot(ns2_ref[...], wl2_ref[...],
                        preferred_element_type=jnp.float32, precision=_DEF)
              + bl_ref[...])
    m = jnp.max(logits, axis=0, keepdims=True)
    e = jnp.exp(logits - m)
    out_ref[...] = e / jnp.sum(e, axis=0, keepdims=True)


_readout = pl.pallas_call(
    _readout_body,
    out_shape=jax.ShapeDtypeStruct((V, 3), jnp.float32),
)


def kernel(X_Node, X_Neis, dg_list, node_features, W_xi, b_xi,
           W_rou, b_rou, W_lin, b_lin):
    xn = X_Node.astype(jnp.int32)
    xw = X_Neis.astype(jnp.int32)
    nf = node_features

    degp = _hist(xn, jnp.zeros((V * 16,), jnp.float32))
    deg16 = _reduce_deg(degp).reshape(V, 16)

    wr_pad = jnp.zeros((LN, LN), jnp.float32).at[:, :S].set(W_rou.T)
    br_pad = jnp.zeros((1, LN), jnp.float32).at[:, :S].set(b_rou)
    P = jnp.zeros((16, LN), jnp.float32).at[0, S].set(1.0)
    BD = _prep(nf, wr_pad, br_pad, deg16, P)

    Xcat, BDg = _gather(xn, xw, nf, BD)

    jp = jnp.arange(S * S)
    perm = (jp % S) * S + jp // S
    r = jnp.arange(S)
    Tm = (jp[None, :] // S == r[:, None]).astype(jnp.float32)
    Tm_pad = jnp.zeros((LN, S * S), jnp.float32).at[:S].set(Tm)
    F = (jnp.arange(LN)[:, None] % S == r[None, :]).astype(jnp.float32)

    Hn = _dense(Xcat, BDg, dg_list.reshape(E, 1),
                W_xi.T[:, perm], b_xi[perm].reshape(1, S * S), Tm_pad, F)

    partials = _scatter(xn, Hn.reshape(E * S), jnp.zeros((V * S,), jnp.float32))
    ns2 = _reduce_part(partials).reshape(V, S)

    out = _readout(nf, ns2, W_lin[:, :LN].T, W_lin[:, LN:].T,
                   b_lin.reshape(1, 3))
    return out


# speedup vs baseline: 3.1843x; 1.0373x over previous
import functools

import jax
import jax.numpy as jnp
from jax import lax
from jax.experimental import pallas as pl
from jax.experimental.pallas import tpu as pltpu
from jax.experimental.pallas import tpu_sc as plsc

V = 2048
LN = 128
S = 32
MU = 0.9
E = 32768

NC = 2
NS = 16
NW = NC * NS
EPW = E // NW
CH = 128
NCH = EPW // CH

_mesh = plsc.VectorSubcoreMesh(core_axis_name="c", subcore_axis_name="s")

_DEF = lax.Precision.DEFAULT
_HI = lax.Precision.HIGHEST


@functools.partial(
    pl.kernel,
    out_type=jax.ShapeDtypeStruct((NW, V * 16), jnp.float32),
    mesh=_mesh,
    scratch_types=[
        pltpu.VMEM((EPW,), jnp.int32),
        pltpu.VMEM((V * 16,), jnp.float32),
    ],
)
def _hist(xn_hbm, zeros_hbm, deg_hbm, idx_v, deg_v):
    cid = lax.axis_index("c")
    sid = lax.axis_index("s")
    wid = sid * NC + cid
    base = wid * EPW
    pltpu.sync_copy(zeros_hbm, deg_v)
    pltpu.sync_copy(xn_hbm.at[pl.ds(base, EPW)], idx_v)

    def group(g, carry):
        uvec = idx_v[pl.ds(g * 16, 16)]
        for l in range(16):
            u = uvec[l]
            dsl = pl.ds(u * 16, 16)
            deg_v[dsl] = deg_v[dsl] + 1.0
        return carry

    lax.fori_loop(0, EPW // 16, group, 0)
    pltpu.sync_copy(deg_v, deg_hbm.at[wid])


def _make_reduce(width, bk):
    def body(in_ref, out_ref):
        out_ref[...] = jnp.sum(in_ref[...], axis=0, keepdims=True)[None]

    return pl.pallas_call(
        body,
        grid=(width // bk,),
        in_specs=[pl.BlockSpec((NW, bk), lambda i: (0, i))],
        out_specs=pl.BlockSpec((1, 1, bk), lambda i: (i, 0, 0)),
        out_shape=jax.ShapeDtypeStruct((width // bk, 1, bk), jnp.float32),
    )


_reduce_part = _make_reduce(V * S, 8192)
_reduce_deg = _make_reduce(V * 16, 8192)


def _prep_body(nf_ref, wr_ref, br_ref, deg16_ref, p_ref, out_ref):
    z = jnp.dot(nf_ref[...], wr_ref[...],
                preferred_element_type=jnp.float32, precision=_DEF)
    out_ref[...] = (jnp.tanh(z + br_ref[...])
                    + jnp.dot(deg16_ref[...], p_ref[...],
                              preferred_element_type=jnp.float32,
                              precision=_HI))


_prep = pl.pallas_call(
    _prep_body,
    out_shape=jax.ShapeDtypeStruct((V, LN), jnp.float32),
)


@functools.partial(
    pl.kernel,
    out_type=(
        jax.ShapeDtypeStruct((E, 2 * LN), jnp.float32),
        jax.ShapeDtypeStruct((E, LN), jnp.float32),
    ),
    mesh=_mesh,
    scratch_types=[
        pltpu.VMEM((EPW,), jnp.int32),
        pltpu.VMEM((EPW,), jnp.int32),
        pltpu.VMEM((2, CH, LN), jnp.float32),
        pltpu.VMEM((2, CH, LN), jnp.float32),
        pltpu.VMEM((2, CH, LN), jnp.float32),
        pltpu.SemaphoreType.DMA,
        pltpu.SemaphoreType.DMA,
        pltpu.SemaphoreType.DMA,
        pltpu.SemaphoreType.DMA,
    ],
)
def _gather(xn_hbm, xw_hbm, nf_hbm, bd_hbm,
            xcat_hbm, bdg_hbm,
            idxn_v, idxw_v, bufn_v, bufw_v, bufb_v,
            gsem0, gsem1, ssem0, ssem1):
    cid = lax.axis_index("c")
    sid = lax.axis_index("s")
    wid = sid * NC + cid
    base = wid * EPW
    gsems = (gsem0, gsem1)
    ssems = (ssem0, ssem1)
    pltpu.sync_copy(xn_hbm.at[pl.ds(base, EPW)], idxn_v)
    pltpu.sync_copy(xw_hbm.at[pl.ds(base, EPW)], idxw_v)
    gathers = [None, None]
    stores = [None, None]
    for k in range(NCH + 1):
        if k < NCH:
            b = k % 2
            if stores[b] is not None:
                for cp in stores[b]:
                    cp.wait()
            sl = pl.ds(k * CH, CH)
            gathers[b] = (
                pltpu.async_copy(nf_hbm.at[idxn_v.at[sl]], bufn_v.at[b], gsems[b]),
                pltpu.async_copy(nf_hbm.at[idxw_v.at[sl]], bufw_v.at[b], gsems[b]),
                pltpu.async_copy(bd_hbm.at[idxn_v.at[sl]], bufb_v.at[b], gsems[b]),
            )
        if k >= 1:
            b2 = (k - 1) % 2
            for cp in gathers[b2]:
                cp.wait()
            off = base + (k - 1) * CH
            stores[b2] = (
                pltpu.async_copy(bufn_v.at[b2],
                                 xcat_hbm.at[pl.ds(off, CH), pl.ds(0, LN)], ssems[b2]),
                pltpu.async_copy(bufw_v.at[b2],
                                 xcat_hbm.at[pl.ds(off, CH), pl.ds(LN, LN)], ssems[b2]),
                pltpu.async_copy(bufb_v.at[b2],
                                 bdg_hbm.at[pl.ds(off, CH)], ssems[b2]),
            )
    for b in range(2):
        if stores[b] is not None:
            for cp in stores[b]:
                cp.wait()


BE = 2048


def _dense_body(xcat_ref, bdg_ref, dg_ref, w1_ref, bxi_ref,
                tm_ref, f_ref, out_ref):
    z = jnp.dot(xcat_ref[...], w1_ref[...],
                preferred_element_type=jnp.float32, precision=_DEF) + bxi_ref[...]
    a = jnp.tanh(z)
    hgt = jnp.dot(bdg_ref[...], tm_ref[...],
                  preferred_element_type=jnp.float32, precision=_DEF)
    p = a * hgt
    acc = p[:, 0:128]
    for k in range(1, 8):
        acc = acc + p[:, 128 * k:128 * (k + 1)]
    core = jnp.dot(acc, f_ref[...],
                   preferred_element_type=jnp.float32, precision=_HI)
    deg = bdg_ref[...][:, S:S + 1]
    hn = core * deg * ((MU / S) / dg_ref[...]) + bdg_ref[...][:, :S]
    out_ref[...] = hn.astype(jnp.bfloat16).astype(jnp.float32)


_dense = pl.pallas_call(
    _dense_body,
    grid=(E // BE,),
    in_specs=[
        pl.BlockSpec((BE, 2 * LN), lambda i: (i, 0)),
        pl.BlockSpec((BE, LN), lambda i: (i, 0)),
        pl.BlockSpec((BE, 1), lambda i: (i, 0)),
        pl.BlockSpec((2 * LN, S * S), lambda i: (0, 0)),
        pl.BlockSpec((1, S * S), lambda i: (0, 0)),
        pl.BlockSpec((LN, S * S), lambda i: (0, 0)),
        pl.BlockSpec((LN, S), lambda i: (0, 0)),
    ],
    out_specs=pl.BlockSpec((BE, S), lambda i: (i, 0)),
    out_shape=jax.ShapeDtypeStruct((E, S), jnp.float32),
)


@functools.partial(
    pl.kernel,
    out_type=jax.ShapeDtypeStruct((NW, V * S), jnp.float32),
    mesh=_mesh,
    scratch_types=[
        pltpu.VMEM((EPW,), jnp.int32),
        pltpu.VMEM((CH * S,), jnp.float32),
        pltpu.VMEM((V * S,), jnp.float32),
    ],
)
def _scatter(xn_hbm, hnf_hbm, zeros_hbm, out_hbm, idx_v, rows_v, acc_v):
    cid = lax.axis_index("c")
    sid = lax.axis_index("s")
    wid = sid * NC + cid
    base = wid * EPW
    pltpu.sync_copy(zeros_hbm, acc_v)
    pltpu.sync_copy(xn_hbm.at[pl.ds(base, EPW)], idx_v)
    for k in range(NCH):
        pltpu.sync_copy(hnf_hbm.at[pl.ds((base + k * CH) * S, CH * S)], rows_v)

        def group(g, carry, k=k):
            uvec = idx_v[pl.ds(k * CH + g * 16, 16)]
            for l in range(16):
                u = uvec[l]
                e = g * 16 + l
                for h in range(S // 16):
                    sl = pl.ds(u * S + h * 16, 16)
                    acc_v[sl] = acc_v[sl] + rows_v[pl.ds(e * S + h * 16, 16)]
            return carry

        lax.fori_loop(0, CH // 16, group, 0)
    pltpu.sync_copy(acc_v, out_hbm.at[wid])


def _readout_body(nf_ref, ns2_ref, wl1_ref, wl2_ref, bl_ref, out_ref):
    logits = (jnp.dot(nf_ref[...], wl1_ref[...],
                      preferred_element_type=jnp.float32, precision=_DEF)
              + jnp.dot(ns2_ref[...], wl2_ref[...],
                        preferred_element_type=jnp.float32, precision=_DEF)
              + bl_ref[...])
    m = jnp.max(logits, axis=0, keepdims=True)
    e = jnp.exp(logits - m)
    out_ref[...] = e / jnp.sum(e, axis=0, keepdims=True)


_readout = pl.pallas_call(
    _readout_body,
    out_shape=jax.ShapeDtypeStruct((V, 3), jnp.float32),
)


def kernel(X_Node, X_Neis, dg_list, node_features, W_xi, b_xi,
           W_rou, b_rou, W_lin, b_lin):
    xn = X_Node.astype(jnp.int32)
    xw = X_Neis.astype(jnp.int32)
    nf = node_features

    degp = _hist(xn, jnp.zeros((V * 16,), jnp.float32))
    deg16 = _reduce_deg(degp).reshape(V, 16)

    wr_pad = jnp.zeros((LN, LN), jnp.float32).at[:, :S].set(W_rou.T)
    br_pad = jnp.zeros((1, LN), jnp.float32).at[:, :S].set(b_rou)
    P = jnp.zeros((16, LN), jnp.float32).at[0, S].set(1.0)
    BD = _prep(nf, wr_pad, br_pad, deg16, P)

    Xcat, BDg = _gather(xn, xw, nf, BD)

    jp = jnp.arange(S * S)
    perm = (jp % S) * S + jp // S
    r = jnp.arange(S)
    Tm = (jp[None, :] // S == r[:, None]).astype(jnp.float32)
    Tm_pad = jnp.zeros((LN, S * S), jnp.float32).at[:S].set(Tm)
    F = (jnp.arange(LN)[:, None] % S == r[None, :]).astype(jnp.float32)

    Hn = _dense(Xcat, BDg, dg_list.reshape(E, 1),
                W_xi.T[:, perm], b_xi[perm].reshape(1, S * S), Tm_pad, F)

    partials = _scatter(xn, Hn.reshape(E * S), jnp.zeros((V * S,), jnp.float32))
    ns2 = _reduce_part(partials).reshape(V, S)

    out = _readout(nf, ns2, W_lin[:, :LN].T, W_lin[:, LN:].T,
                   b_lin.reshape(1, 3))
    return out


# speedup vs baseline: 3.4483x; 1.0829x over previous
import functools

import jax
import jax.numpy as jnp
from jax import lax
from jax.experimental import pallas as pl
from jax.experimental.pallas import tpu as pltpu
from jax.experimental.pallas import tpu_sc as plsc

V = 2048
LN = 128
S = 32
MU = 0.9
E = 32768

NC = 2
NS = 16
NW = NC * NS
EPW = E // NW
CH = 128
NCH = EPW // CH

_mesh = plsc.VectorSubcoreMesh(core_axis_name="c", subcore_axis_name="s")

_DEF = lax.Precision.DEFAULT
_HI = lax.Precision.HIGHEST


@functools.partial(
    pl.kernel,
    out_type=jax.ShapeDtypeStruct((NW, V * 16), jnp.float32),
    mesh=_mesh,
    scratch_types=[
        pltpu.VMEM((EPW,), jnp.int32),
        pltpu.VMEM((V * 16,), jnp.float32),
    ],
)
def _hist(xn_hbm, zeros_hbm, deg_hbm, idx_v, deg_v):
    cid = lax.axis_index("c")
    sid = lax.axis_index("s")
    wid = sid * NC + cid
    base = wid * EPW
    pltpu.sync_copy(zeros_hbm, deg_v)
    pltpu.sync_copy(xn_hbm.at[pl.ds(base, EPW)], idx_v)

    def group(g, carry):
        uvec = idx_v[pl.ds(g * 16, 16)]
        for l in range(16):
            u = uvec[l]
            dsl = pl.ds(u * 16, 16)
            deg_v[dsl] = deg_v[dsl] + 1.0
        return carry

    lax.fori_loop(0, EPW // 16, group, 0)
    pltpu.sync_copy(deg_v, deg_hbm.at[wid])


def _make_reduce(width, bk):
    def body(in_ref, out_ref):
        out_ref[...] = jnp.sum(in_ref[...], axis=0, keepdims=True)[None]

    return pl.pallas_call(
        body,
        grid=(width // bk,),
        in_specs=[pl.BlockSpec((NW, bk), lambda i: (0, i))],
        out_specs=pl.BlockSpec((1, 1, bk), lambda i: (i, 0, 0)),
        out_shape=jax.ShapeDtypeStruct((width // bk, 1, bk), jnp.float32),
    )


_reduce_part = _make_reduce(V * S, 8192)
_reduce_deg = _make_reduce(V * 16, 8192)


def _prep_body(nf_ref, wr_ref, br_ref, deg16_ref, p_ref, out_ref):
    z = jnp.dot(nf_ref[...], wr_ref[...],
                preferred_element_type=jnp.float32, precision=_DEF)
    out_ref[...] = (jnp.tanh(z + br_ref[...])
                    + jnp.dot(deg16_ref[...], p_ref[...],
                              preferred_element_type=jnp.float32,
                              precision=_HI))


_prep = pl.pallas_call(
    _prep_body,
    out_shape=jax.ShapeDtypeStruct((V, LN), jnp.float32),
)


@functools.partial(
    pl.kernel,
    out_type=(
        jax.ShapeDtypeStruct((E, 2 * LN), jnp.float32),
        jax.ShapeDtypeStruct((E, LN), jnp.float32),
    ),
    mesh=_mesh,
    scratch_types=[
        pltpu.VMEM((EPW,), jnp.int32),
        pltpu.VMEM((EPW,), jnp.int32),
        pltpu.VMEM((2, CH, LN), jnp.float32),
        pltpu.VMEM((2, CH, LN), jnp.float32),
        pltpu.VMEM((2, CH, LN), jnp.float32),
        pltpu.SemaphoreType.DMA,
        pltpu.SemaphoreType.DMA,
        pltpu.SemaphoreType.DMA,
        pltpu.SemaphoreType.DMA,
    ],
)
def _gather(xn_hbm, xw_hbm, nf_hbm, bd_hbm,
            xcat_hbm, bdg_hbm,
            idxn_v, idxw_v, bufn_v, bufw_v, bufb_v,
            gsem0, gsem1, ssem0, ssem1):
    cid = lax.axis_index("c")
    sid = lax.axis_index("s")
    wid = sid * NC + cid
    base = wid * EPW
    gsems = (gsem0, gsem1)
    ssems = (ssem0, ssem1)
    pltpu.sync_copy(xn_hbm.at[pl.ds(base, EPW)], idxn_v)
    pltpu.sync_copy(xw_hbm.at[pl.ds(base, EPW)], idxw_v)
    gathers = [None, None]
    stores = [None, None]
    for k in range(NCH + 1):
        if k < NCH:
            b = k % 2
            if stores[b] is not None:
                for cp in stores[b]:
                    cp.wait()
            sl = pl.ds(k * CH, CH)
            gathers[b] = (
                pltpu.async_copy(nf_hbm.at[idxn_v.at[sl]], bufn_v.at[b], gsems[b]),
                pltpu.async_copy(nf_hbm.at[idxw_v.at[sl]], bufw_v.at[b], gsems[b]),
                pltpu.async_copy(bd_hbm.at[idxn_v.at[sl]], bufb_v.at[b], gsems[b]),
            )
        if k >= 1:
            b2 = (k - 1) % 2
            for cp in gathers[b2]:
                cp.wait()
            off = base + (k - 1) * CH
            stores[b2] = (
                pltpu.async_copy(bufn_v.at[b2],
                                 xcat_hbm.at[pl.ds(off, CH), pl.ds(0, LN)], ssems[b2]),
                pltpu.async_copy(bufw_v.at[b2],
                                 xcat_hbm.at[pl.ds(off, CH), pl.ds(LN, LN)], ssems[b2]),
                pltpu.async_copy(bufb_v.at[b2],
                                 bdg_hbm.at[pl.ds(off, CH)], ssems[b2]),
            )
    for b in range(2):
        if stores[b] is not None:
            for cp in stores[b]:
                cp.wait()


BE = 2048


def _dense_body(xcat_ref, bdg_ref, dg_ref, w1_ref, bxi_ref,
                tm_ref, f_ref, out_ref):
    z = jnp.dot(xcat_ref[...], w1_ref[...],
                preferred_element_type=jnp.float32, precision=_DEF) + bxi_ref[...]
    a = jnp.tanh(z)
    hgt = jnp.dot(bdg_ref[...], tm_ref[...],
                  preferred_element_type=jnp.float32, precision=_DEF)
    p = a * hgt
    acc = p[:, 0:128]
    for k in range(1, 8):
        acc = acc + p[:, 128 * k:128 * (k + 1)]
    core = jnp.dot(acc, f_ref[...],
                   preferred_element_type=jnp.float32, precision=_HI)
    deg = bdg_ref[...][:, S:S + 1]
    hn = core * deg * ((MU / S) / dg_ref[...]) + bdg_ref[...][:, :S]
    out_ref[...] = hn.astype(jnp.bfloat16).astype(jnp.float32)


_dense = pl.pallas_call(
    _dense_body,
    grid=(E // BE,),
    in_specs=[
        pl.BlockSpec((BE, 2 * LN), lambda i: (i, 0)),
        pl.BlockSpec((BE, LN), lambda i: (i, 0)),
        pl.BlockSpec((BE, 1), lambda i: (i, 0)),
        pl.BlockSpec((2 * LN, S * S), lambda i: (0, 0)),
        pl.BlockSpec((1, S * S), lambda i: (0, 0)),
        pl.BlockSpec((LN, S * S), lambda i: (0, 0)),
        pl.BlockSpec((LN, S), lambda i: (0, 0)),
    ],
    out_specs=pl.BlockSpec((BE, S), lambda i: (i, 0)),
    out_shape=jax.ShapeDtypeStruct((E, S), jnp.float32),
)


@functools.partial(
    pl.kernel,
    out_type=jax.ShapeDtypeStruct((NW, V * S), jnp.float32),
    mesh=_mesh,
    scratch_types=[
        pltpu.VMEM((EPW,), jnp.int32),
        pltpu.VMEM((2, CH, S), jnp.float32),
        pltpu.VMEM((V * S,), jnp.float32),
        pltpu.SemaphoreType.DMA,
        pltpu.SemaphoreType.DMA,
    ],
)
def _scatter(xn_hbm, hn_hbm, zeros_hbm, out_hbm, idx_v, rows_v, acc_v,
             rsem0, rsem1):
    cid = lax.axis_index("c")
    sid = lax.axis_index("s")
    wid = sid * NC + cid
    base = wid * EPW
    rsems = (rsem0, rsem1)
    pltpu.sync_copy(zeros_hbm, acc_v)
    pltpu.sync_copy(xn_hbm.at[pl.ds(base, EPW)], idx_v)
    loads = [None, None]
    loads[0] = pltpu.async_copy(hn_hbm.at[pl.ds(base, CH)], rows_v.at[0], rsems[0])
    for k in range(NCH):
        b = k % 2
        loads[b].wait()
        if k + 1 < NCH:
            b2 = (k + 1) % 2
            loads[b2] = pltpu.async_copy(
                hn_hbm.at[pl.ds(base + (k + 1) * CH, CH)], rows_v.at[b2], rsems[b2])

        def group(g, carry, k=k, b=b):
            uvec = idx_v[pl.ds(k * CH + g * 16, 16)]
            for l in range(16):
                u = uvec[l]
                e = g * 16 + l
                for h in range(S // 16):
                    sl = pl.ds(u * S + h * 16, 16)
                    acc_v[sl] = acc_v[sl] + rows_v[b, e, pl.ds(h * 16, 16)]
            return carry

        lax.fori_loop(0, CH // 16, group, 0)
    pltpu.sync_copy(acc_v, out_hbm.at[wid])


def _readout_body(nf_ref, ns2_ref, wl1_ref, wl2_ref, bl_ref, out_ref):
    logits = (jnp.dot(nf_ref[...], wl1_ref[...],
                      preferred_element_type=jnp.float32, precision=_DEF)
              + jnp.dot(ns2_ref[...], wl2_ref[...],
                        preferred_element_type=jnp.float32, precision=_DEF)
              + bl_ref[...])
    m = jnp.max(logits, axis=0, keepdims=True)
    e = jnp.exp(logits - m)
    out_ref[...] = e / jnp.sum(e, axis=0, keepdims=True)


_readout = pl.pallas_call(
    _readout_body,
    out_shape=jax.ShapeDtypeStruct((V, 3), jnp.float32),
)


def kernel(X_Node, X_Neis, dg_list, node_features, W_xi, b_xi,
           W_rou, b_rou, W_lin, b_lin):
    xn = X_Node.astype(jnp.int32)
    xw = X_Neis.astype(jnp.int32)
    nf = node_features

    degp = _hist(xn, jnp.zeros((V * 16,), jnp.float32))
    deg16 = _reduce_deg(degp).reshape(V, 16)

    wr_pad = jnp.zeros((LN, LN), jnp.float32).at[:, :S].set(W_rou.T)
    br_pad = jnp.zeros((1, LN), jnp.float32).at[:, :S].set(b_rou)
    P = jnp.zeros((16, LN), jnp.float32).at[0, S].set(1.0)
    BD = _prep(nf, wr_pad, br_pad, deg16, P)

    Xcat, BDg = _gather(xn, xw, nf, BD)

    jp = jnp.arange(S * S)
    perm = (jp % S) * S + jp // S
    r = jnp.arange(S)
    Tm = (jp[None, :] // S == r[:, None]).astype(jnp.float32)
    Tm_pad = jnp.zeros((LN, S * S), jnp.float32).at[:S].set(Tm)
    F = (jnp.arange(LN)[:, None] % S == r[None, :]).astype(jnp.float32)

    Hn = _dense(Xcat, BDg, dg_list.reshape(E, 1),
                W_xi.T[:, perm], b_xi[perm].reshape(1, S * S), Tm_pad, F)

    partials = _scatter(xn, Hn, jnp.zeros((V * S,), jnp.float32))
    ns2 = _reduce_part(partials).reshape(V, S)

    out = _readout(nf, ns2, W_lin[:, :LN].T, W_lin[:, LN:].T,
                   b_lin.reshape(1, 3))
    return out


# speedup vs baseline: 3.5223x; 1.0215x over previous
import functools

import jax
import jax.numpy as jnp
from jax import lax
from jax.experimental import pallas as pl
from jax.experimental.pallas import tpu as pltpu
from jax.experimental.pallas import tpu_sc as plsc

V = 2048
LN = 128
S = 32
MU = 0.9
E = 32768

NC = 2
NS = 16
NW = NC * NS
EPW = E // NW
CH = 128
NCH = EPW // CH

_mesh = plsc.VectorSubcoreMesh(core_axis_name="c", subcore_axis_name="s")

_DEF = lax.Precision.DEFAULT
_HI = lax.Precision.HIGHEST


@functools.partial(
    pl.kernel,
    out_type=jax.ShapeDtypeStruct((NW, V * 16), jnp.float32),
    mesh=_mesh,
    scratch_types=[
        pltpu.VMEM((EPW,), jnp.int32),
        pltpu.VMEM((V * 16,), jnp.float32),
    ],
)
def _hist(xn_hbm, zeros_hbm, deg_hbm, idx_v, deg_v):
    cid = lax.axis_index("c")
    sid = lax.axis_index("s")
    wid = sid * NC + cid
    base = wid * EPW
    pltpu.sync_copy(zeros_hbm, deg_v)
    pltpu.sync_copy(xn_hbm.at[pl.ds(base, EPW)], idx_v)

    def group(g, carry):
        uvec = idx_v[pl.ds(g * 16, 16)]
        for l in range(16):
            u = uvec[l]
            dsl = pl.ds(u * 16, 16)
            deg_v[dsl] = deg_v[dsl] + 1.0
        return carry

    lax.fori_loop(0, EPW // 16, group, 0)
    pltpu.sync_copy(deg_v, deg_hbm.at[wid])


def _make_reduce(width, bk):
    def body(in_ref, out_ref):
        out_ref[...] = jnp.sum(in_ref[...], axis=0, keepdims=True)[None]

    return pl.pallas_call(
        body,
        grid=(width // bk,),
        in_specs=[pl.BlockSpec((NW, bk), lambda i: (0, i))],
        out_specs=pl.BlockSpec((1, 1, bk), lambda i: (i, 0, 0)),
        out_shape=jax.ShapeDtypeStruct((width // bk, 1, bk), jnp.float32),
    )


_reduce_deg = _make_reduce(V * 16, 8192)


def _reduce2_body(a_ref, b_ref, out_ref):
    out_ref[...] = (jnp.sum(a_ref[...], axis=0, keepdims=True)
                    + jnp.sum(b_ref[...], axis=0, keepdims=True))[None]


_reduce_part2 = pl.pallas_call(
    _reduce2_body,
    grid=(V * S // 8192,),
    in_specs=[pl.BlockSpec((NW, 8192), lambda i: (0, i)),
              pl.BlockSpec((NW, 8192), lambda i: (0, i))],
    out_specs=pl.BlockSpec((1, 1, 8192), lambda i: (i, 0, 0)),
    out_shape=jax.ShapeDtypeStruct((V * S // 8192, 1, 8192), jnp.float32),
)


def _prep_body(nf_ref, wr_ref, br_ref, deg16_ref, p_ref, out_ref):
    z = jnp.dot(nf_ref[...], wr_ref[...],
                preferred_element_type=jnp.float32, precision=_DEF)
    out_ref[...] = (jnp.tanh(z + br_ref[...])
                    + jnp.dot(deg16_ref[...], p_ref[...],
                              preferred_element_type=jnp.float32,
                              precision=_HI))


_prep = pl.pallas_call(
    _prep_body,
    out_shape=jax.ShapeDtypeStruct((V, LN), jnp.float32),
)


def _make_gather(ne, eoff):
    epw = ne // NW
    nch = epw // CH

    @functools.partial(
        pl.kernel,
        out_type=(
            jax.ShapeDtypeStruct((ne, 2 * LN), jnp.float32),
            jax.ShapeDtypeStruct((ne, LN), jnp.float32),
        ),
        mesh=_mesh,
        scratch_types=[
            pltpu.VMEM((epw,), jnp.int32),
            pltpu.VMEM((epw,), jnp.int32),
            pltpu.VMEM((2, CH, LN), jnp.float32),
            pltpu.VMEM((2, CH, LN), jnp.float32),
            pltpu.VMEM((2, CH, LN), jnp.float32),
            pltpu.SemaphoreType.DMA,
            pltpu.SemaphoreType.DMA,
            pltpu.SemaphoreType.DMA,
            pltpu.SemaphoreType.DMA,
        ],
    )
    def gather(xn_hbm, xw_hbm, nf_hbm, bd_hbm,
               xcat_hbm, bdg_hbm,
               idxn_v, idxw_v, bufn_v, bufw_v, bufb_v,
               gsem0, gsem1, ssem0, ssem1):
        cid = lax.axis_index("c")
        sid = lax.axis_index("s")
        wid = sid * NC + cid
        base = wid * epw
        gsems = (gsem0, gsem1)
        ssems = (ssem0, ssem1)
        pltpu.sync_copy(xn_hbm.at[pl.ds(eoff + base, epw)], idxn_v)
        pltpu.sync_copy(xw_hbm.at[pl.ds(eoff + base, epw)], idxw_v)
        gathers = [None, None]
        stores = [None, None]
        for k in range(nch + 1):
            if k < nch:
                b = k % 2
                if stores[b] is not None:
                    for cp in stores[b]:
                        cp.wait()
                sl = pl.ds(k * CH, CH)
                gathers[b] = (
                    pltpu.async_copy(nf_hbm.at[idxn_v.at[sl]], bufn_v.at[b], gsems[b]),
                    pltpu.async_copy(nf_hbm.at[idxw_v.at[sl]], bufw_v.at[b], gsems[b]),
                    pltpu.async_copy(bd_hbm.at[idxn_v.at[sl]], bufb_v.at[b], gsems[b]),
                )
            if k >= 1:
                b2 = (k - 1) % 2
                for cp in gathers[b2]:
                    cp.wait()
                off = base + (k - 1) * CH
                stores[b2] = (
                    pltpu.async_copy(bufn_v.at[b2],
                                     xcat_hbm.at[pl.ds(off, CH), pl.ds(0, LN)], ssems[b2]),
                    pltpu.async_copy(bufw_v.at[b2],
                                     xcat_hbm.at[pl.ds(off, CH), pl.ds(LN, LN)], ssems[b2]),
                    pltpu.async_copy(bufb_v.at[b2],
                                     bdg_hbm.at[pl.ds(off, CH)], ssems[b2]),
                )
        for b in range(2):
            if stores[b] is not None:
                for cp in stores[b]:
                    cp.wait()

    return gather


EH = E // 2
_gather_h = (_make_gather(EH, 0), _make_gather(EH, EH))


BE = 2048


def _dense_body(xcat_ref, bdg_ref, dg_ref, w1_ref, bxi_ref,
                tm_ref, f_ref, out_ref):
    z = jnp.dot(xcat_ref[...], w1_ref[...],
                preferred_element_type=jnp.float32, precision=_DEF) + bxi_ref[...]
    a = jnp.tanh(z)
    hgt = jnp.dot(bdg_ref[...], tm_ref[...],
                  preferred_element_type=jnp.float32, precision=_DEF)
    p = a * hgt
    acc = p[:, 0:128]
    for k in range(1, 8):
        acc = acc + p[:, 128 * k:128 * (k + 1)]
    core = jnp.dot(acc, f_ref[...],
                   preferred_element_type=jnp.float32, precision=_HI)
    deg = bdg_ref[...][:, S:S + 1]
    hn = core * deg * ((MU / S) / dg_ref[...]) + bdg_ref[...][:, :S]
    out_ref[...] = hn.astype(jnp.bfloat16).astype(jnp.float32)


_dense = pl.pallas_call(
    _dense_body,
    grid=(EH // BE,),
    in_specs=[
        pl.BlockSpec((BE, 2 * LN), lambda i: (i, 0)),
        pl.BlockSpec((BE, LN), lambda i: (i, 0)),
        pl.BlockSpec((BE, 1), lambda i: (i, 0)),
        pl.BlockSpec((2 * LN, S * S), lambda i: (0, 0)),
        pl.BlockSpec((1, S * S), lambda i: (0, 0)),
        pl.BlockSpec((LN, S * S), lambda i: (0, 0)),
        pl.BlockSpec((LN, S), lambda i: (0, 0)),
    ],
    out_specs=pl.BlockSpec((BE, S), lambda i: (i, 0)),
    out_shape=jax.ShapeDtypeStruct((EH, S), jnp.float32),
)


def _make_scatter(ne, eoff):
    epw = ne // NW
    nch = epw // CH

    @functools.partial(
        pl.kernel,
        out_type=jax.ShapeDtypeStruct((NW, V * S), jnp.float32),
        mesh=_mesh,
        scratch_types=[
            pltpu.VMEM((epw,), jnp.int32),
            pltpu.VMEM((2, CH, S), jnp.float32),
            pltpu.VMEM((V * S,), jnp.float32),
            pltpu.SemaphoreType.DMA,
            pltpu.SemaphoreType.DMA,
        ],
    )
    def scatter(xn_hbm, hn_hbm, zeros_hbm, out_hbm, idx_v, rows_v, acc_v,
                rsem0, rsem1):
        cid = lax.axis_index("c")
        sid = lax.axis_index("s")
        wid = sid * NC + cid
        base = wid * epw
        rsems = (rsem0, rsem1)
        pltpu.sync_copy(zeros_hbm, acc_v)
        pltpu.sync_copy(xn_hbm.at[pl.ds(eoff + base, epw)], idx_v)
        loads = [None, None]
        loads[0] = pltpu.async_copy(hn_hbm.at[pl.ds(base, CH)], rows_v.at[0], rsems[0])
        for k in range(nch):
            b = k % 2
            loads[b].wait()
            if k + 1 < nch:
                b2 = (k + 1) % 2
                loads[b2] = pltpu.async_copy(
                    hn_hbm.at[pl.ds(base + (k + 1) * CH, CH)], rows_v.at[b2], rsems[b2])

            def group(g, carry, k=k, b=b):
                uvec = idx_v[pl.ds(k * CH + g * 16, 16)]
                for l in range(16):
                    u = uvec[l]
                    e = g * 16 + l
                    for h in range(S // 16):
                        sl = pl.ds(u * S + h * 16, 16)
                        acc_v[sl] = acc_v[sl] + rows_v[b, e, pl.ds(h * 16, 16)]
                return carry

            lax.fori_loop(0, CH // 16, group, 0)
        pltpu.sync_copy(acc_v, out_hbm.at[wid])

    return scatter


_scatter_h = (_make_scatter(EH, 0), _make_scatter(EH, EH))


def _readout_body(nf_ref, ns2_ref, wl1_ref, wl2_ref, bl_ref, out_ref):
    logits = (jnp.dot(nf_ref[...], wl1_ref[...],
                      preferred_element_type=jnp.float32, precision=_DEF)
              + jnp.dot(ns2_ref[...], wl2_ref[...],
                        preferred_element_type=jnp.float32, precision=_DEF)
              + bl_ref[...])
    m = jnp.max(logits, axis=0, keepdims=True)
    e = jnp.exp(logits - m)
    out_ref[...] = e / jnp.sum(e, axis=0, keepdims=True)


_readout = pl.pallas_call(
    _readout_body,
    out_shape=jax.ShapeDtypeStruct((V, 3), jnp.float32),
)


def kernel(X_Node, X_Neis, dg_list, node_features, W_xi, b_xi,
           W_rou, b_rou, W_lin, b_lin):
    xn = X_Node.astype(jnp.int32)
    xw = X_Neis.astype(jnp.int32)
    nf = node_features

    degp = _hist(xn, jnp.zeros((V * 16,), jnp.float32))
    deg16 = _reduce_deg(degp).reshape(V, 16)

    wr_pad = jnp.zeros((LN, LN), jnp.float32).at[:, :S].set(W_rou.T)
    br_pad = jnp.zeros((1, LN), jnp.float32).at[:, :S].set(b_rou)
    P = jnp.zeros((16, LN), jnp.float32).at[0, S].set(1.0)
    BD = _prep(nf, wr_pad, br_pad, deg16, P)

    Xcat0, BDg0 = _gather_h[0](xn, xw, nf, BD)
    Xcat1, BDg1 = _gather_h[1](xn, xw, nf, BD)

    jp = jnp.arange(S * S)
    perm = (jp % S) * S + jp // S
    r = jnp.arange(S)
    Tm = (jp[None, :] // S == r[:, None]).astype(jnp.float32)
    Tm_pad = jnp.zeros((LN, S * S), jnp.float32).at[:S].set(Tm)
    F = (jnp.arange(LN)[:, None] % S == r[None, :]).astype(jnp.float32)

    dg2 = dg_list.reshape(E, 1)
    w1p = W_xi.T[:, perm]
    bxp = b_xi[perm].reshape(1, S * S)
    Hn0 = _dense(Xcat0, BDg0, dg2[:EH], w1p, bxp, Tm_pad, F)
    Hn1 = _dense(Xcat1, BDg1, dg2[EH:], w1p, bxp, Tm_pad, F)

    zvs = jnp.zeros((V * S,), jnp.float32)
    p0 = _scatter_h[0](xn, Hn0, zvs)
    p1 = _scatter_h[1](xn, Hn1, zvs)
    ns2 = _reduce_part2(p0, p1).reshape(V, S)

    out = _readout(nf, ns2, W_lin[:, :LN].T, W_lin[:, LN:].T,
                   b_lin.reshape(1, 3))
    return out


# speedup vs baseline: 3.5264x; 1.0012x over previous
import functools

import jax
import jax.numpy as jnp
from jax import lax
from jax.experimental import pallas as pl
from jax.experimental.pallas import tpu as pltpu
from jax.experimental.pallas import tpu_sc as plsc

V = 2048
LN = 128
S = 32
MU = 0.9
E = 32768

NC = 2
NS = 16
NW = NC * NS
EPW = E // NW
CH = 128

_mesh = plsc.VectorSubcoreMesh(core_axis_name="c", subcore_axis_name="s")

_DEF = lax.Precision.DEFAULT
_HI = lax.Precision.HIGHEST


@functools.partial(
    pl.kernel,
    out_type=jax.ShapeDtypeStruct((NW, V * 16), jnp.float32),
    mesh=_mesh,
    scratch_types=[
        pltpu.VMEM((EPW,), jnp.int32),
        pltpu.VMEM((V * 16,), jnp.float32),
    ],
)
def _hist(xn_hbm, zeros_hbm, deg_hbm, idx_v, deg_v):
    cid = lax.axis_index("c")
    sid = lax.axis_index("s")
    wid = sid * NC + cid
    base = wid * EPW
    pltpu.sync_copy(zeros_hbm, deg_v)
    pltpu.sync_copy(xn_hbm.at[pl.ds(base, EPW)], idx_v)

    def group(g, carry):
        uvec = idx_v[pl.ds(g * 16, 16)]
        for l in range(16):
            u = uvec[l]
            dsl = pl.ds(u * 16, 16)
            deg_v[dsl] = deg_v[dsl] + 1.0
        return carry

    lax.fori_loop(0, EPW // 16, group, 0)
    pltpu.sync_copy(deg_v, deg_hbm.at[wid])


def _make_reduce(width, bk):
    def body(in_ref, out_ref):
        out_ref[...] = jnp.sum(in_ref[...], axis=0, keepdims=True)[None]

    return pl.pallas_call(
        body,
        grid=(width // bk,),
        in_specs=[pl.BlockSpec((NW, bk), lambda i: (0, i))],
        out_specs=pl.BlockSpec((1, 1, bk), lambda i: (i, 0, 0)),
        out_shape=jax.ShapeDtypeStruct((width // bk, 1, bk), jnp.float32),
    )


_reduce_deg = _make_reduce(V * 16, 8192)


def _reduce2_body(a_ref, b_ref, out_ref):
    out_ref[...] = (jnp.sum(a_ref[...], axis=0, keepdims=True)
                    + jnp.sum(b_ref[...], axis=0, keepdims=True))[None]


_reduce_part2 = pl.pallas_call(
    _reduce2_body,
    grid=(V * S // 8192,),
    in_specs=[pl.BlockSpec((NW, 8192), lambda i: (0, i)),
              pl.BlockSpec((NW, 8192), lambda i: (0, i))],
    out_specs=pl.BlockSpec((1, 1, 8192), lambda i: (i, 0, 0)),
    out_shape=jax.ShapeDtypeStruct((V * S // 8192, 1, 8192), jnp.float32),
)


def _prep_body(nf_ref, wr_ref, br_ref, deg16_ref, p_ref, out_ref):
    z = jnp.dot(nf_ref[...], wr_ref[...],
                preferred_element_type=jnp.float32, precision=_DEF)
    out_ref[...] = (jnp.tanh(z + br_ref[...])
                    + jnp.dot(deg16_ref[...], p_ref[...],
                              preferred_element_type=jnp.float32,
                              precision=_HI))


_prep = pl.pallas_call(
    _prep_body,
    out_shape=jax.ShapeDtypeStruct((V, LN), jnp.float32),
)


def _make_gather(ne, eoff):
    epw = ne // NW
    nch = epw // CH

    @functools.partial(
        pl.kernel,
        out_type=(
            jax.ShapeDtypeStruct((ne, 2 * LN), jnp.float32),
            jax.ShapeDtypeStruct((ne, LN), jnp.float32),
        ),
        mesh=_mesh,
        scratch_types=[
            pltpu.VMEM((epw,), jnp.int32),
            pltpu.VMEM((epw,), jnp.int32),
            pltpu.VMEM((2, CH, LN), jnp.float32),
            pltpu.VMEM((2, CH, LN), jnp.float32),
            pltpu.VMEM((2, CH, LN), jnp.float32),
            pltpu.SemaphoreType.DMA,
            pltpu.SemaphoreType.DMA,
            pltpu.SemaphoreType.DMA,
            pltpu.SemaphoreType.DMA,
        ],
    )
    def gather(xn_hbm, xw_hbm, nf_hbm, bd_hbm,
               xcat_hbm, bdg_hbm,
               idxn_v, idxw_v, bufn_v, bufw_v, bufb_v,
               gsem0, gsem1, ssem0, ssem1):
        cid = lax.axis_index("c")
        sid = lax.axis_index("s")
        wid = sid * NC + cid
        base = wid * epw
        gsems = (gsem0, gsem1)
        ssems = (ssem0, ssem1)
        pltpu.sync_copy(xn_hbm.at[pl.ds(eoff + base, epw)], idxn_v)
        pltpu.sync_copy(xw_hbm.at[pl.ds(eoff + base, epw)], idxw_v)
        gathers = [None, None]
        stores = [None, None]
        for k in range(nch + 1):
            if k < nch:
                b = k % 2
                if stores[b] is not None:
                    for cp in stores[b]:
                        cp.wait()
                sl = pl.ds(k * CH, CH)
                gathers[b] = (
                    pltpu.async_copy(nf_hbm.at[idxn_v.at[sl]], bufn_v.at[b], gsems[b]),
                    pltpu.async_copy(nf_hbm.at[idxw_v.at[sl]], bufw_v.at[b], gsems[b]),
                    pltpu.async_copy(bd_hbm.at[idxn_v.at[sl]], bufb_v.at[b], gsems[b]),
                )
            if k >= 1:
                b2 = (k - 1) % 2
                for cp in gathers[b2]:
                    cp.wait()
                off = base + (k - 1) * CH
                stores[b2] = (
                    pltpu.async_copy(bufn_v.at[b2],
                                     xcat_hbm.at[pl.ds(off, CH), pl.ds(0, LN)], ssems[b2]),
                    pltpu.async_copy(bufw_v.at[b2],
                                     xcat_hbm.at[pl.ds(off, CH), pl.ds(LN, LN)], ssems[b2]),
                    pltpu.async_copy(bufb_v.at[b2],
                                     bdg_hbm.at[pl.ds(off, CH)], ssems[b2]),
                )
        for b in range(2):
            if stores[b] is not None:
                for cp in stores[b]:
                    cp.wait()

    return gather


EH = E // 2
_gather_h = (_make_gather(EH, 0), _make_gather(EH, EH))


BE = 2048


def _dense_body(xcat_ref, bdg_ref, dg_ref, w1_ref, bxi_ref,
                tm_ref, f_ref, out_ref):
    z = jnp.dot(xcat_ref[...], w1_ref[...],
                preferred_element_type=jnp.float32, precision=_DEF) + bxi_ref[...]
    a = jnp.tanh(z)
    hgt = jnp.dot(bdg_ref[...], tm_ref[...],
                  preferred_element_type=jnp.float32, precision=_DEF)
    p = a * hgt
    acc = p[:, 0:128]
    for k in range(1, 8):
        acc = acc + p[:, 128 * k:128 * (k + 1)]
    core = jnp.dot(acc, f_ref[...],
                   preferred_element_type=jnp.float32, precision=_HI)
    deg = bdg_ref[...][:, S:S + 1]
    hn = core * deg * ((MU / S) / dg_ref[...]) + bdg_ref[...][:, :S]
    out_ref[...] = hn.astype(jnp.bfloat16).astype(jnp.float32)


_dense = pl.pallas_call(
    _dense_body,
    grid=(EH // BE,),
    in_specs=[
        pl.BlockSpec((BE, 2 * LN), lambda i: (i, 0)),
        pl.BlockSpec((BE, LN), lambda i: (i, 0)),
        pl.BlockSpec((BE, 1), lambda i: (i, 0)),
        pl.BlockSpec((2 * LN, S * S), lambda i: (0, 0)),
        pl.BlockSpec((1, S * S), lambda i: (0, 0)),
        pl.BlockSpec((LN, S * S), lambda i: (0, 0)),
        pl.BlockSpec((LN, S), lambda i: (0, 0)),
    ],
    out_specs=pl.BlockSpec((BE, S), lambda i: (i, 0)),
    out_shape=jax.ShapeDtypeStruct((EH, S), jnp.float32),
)


def _make_scatter(ne, eoff):
    epw = ne // NW
    nch = epw // CH

    @functools.partial(
        pl.kernel,
        out_type=jax.ShapeDtypeStruct((NW, V * S), jnp.float32),
        mesh=_mesh,
        scratch_types=[
            pltpu.VMEM((epw,), jnp.int32),
            pltpu.VMEM((2, CH, S), jnp.float32),
            pltpu.VMEM((V * S,), jnp.float32),
            pltpu.SemaphoreType.DMA,
            pltpu.SemaphoreType.DMA,
        ],
    )
    def scatter(xn_hbm, hn_hbm, zeros_hbm, out_hbm, idx_v, rows_v, acc_v,
                rsem0, rsem1):
        cid = lax.axis_index("c")
        sid = lax.axis_index("s")
        wid = sid * NC + cid
        base = wid * epw
        rsems = (rsem0, rsem1)
        pltpu.sync_copy(zeros_hbm, acc_v)
        pltpu.sync_copy(xn_hbm.at[pl.ds(eoff + base, epw)], idx_v)
        loads = [None, None]
        loads[0] = pltpu.async_copy(hn_hbm.at[pl.ds(base, CH)], rows_v.at[0], rsems[0])
        for k in range(nch):
            b = k % 2
            loads[b].wait()
            if k + 1 < nch:
                b2 = (k + 1) % 2
                loads[b2] = pltpu.async_copy(
                    hn_hbm.at[pl.ds(base + (k + 1) * CH, CH)], rows_v.at[b2], rsems[b2])

            def group(g, carry, k=k, b=b):
                uvec = idx_v[pl.ds(k * CH + g * 16, 16)]
                for l in range(16):
                    u = uvec[l]
                    e = g * 16 + l
                    for h in range(S // 16):
                        sl = pl.ds(u * S + h * 16, 16)
                        acc_v[sl] = acc_v[sl] + rows_v[b, e, pl.ds(h * 16, 16)]
                return carry

            lax.fori_loop(0, CH // 16, group, 0)
        pltpu.sync_copy(acc_v, out_hbm.at[wid])

    return scatter


_scatter_h = (_make_scatter(EH, 0), _make_scatter(EH, EH))


def _readout_body(nf_ref, ns2_ref, wl1_ref, wl2_ref, bl_ref, out_ref):
    logits = (jnp.dot(nf_ref[...], wl1_ref[...],
                      preferred_element_type=jnp.float32, precision=_DEF)
              + jnp.dot(ns2_ref[...], wl2_ref[...],
                        preferred_element_type=jnp.float32, precision=_DEF)
              + bl_ref[...])
    m = jnp.max(logits, axis=0, keepdims=True)
    e = jnp.exp(logits - m)
    out_ref[...] = e / jnp.sum(e, axis=0, keepdims=True)


_readout = pl.pallas_call(
    _readout_body,
    out_shape=jax.ShapeDtypeStruct((V, 3), jnp.float32),
)


def kernel(X_Node, X_Neis, dg_list, node_features, W_xi, b_xi,
           W_rou, b_rou, W_lin, b_lin):
    xn = X_Node.astype(jnp.int32)
    xw = X_Neis.astype(jnp.int32)
    nf = node_features

    degp = _hist(xn, jnp.zeros((V * 16,), jnp.float32))
    deg16 = _reduce_deg(degp).reshape(V, 16)

    wr_pad = jnp.zeros((LN, LN), jnp.float32).at[:, :S].set(W_rou.T)
    br_pad = jnp.zeros((1, LN), jnp.float32).at[:, :S].set(b_rou)
    P = jnp.zeros((16, LN), jnp.float32).at[0, S].set(1.0)
    BD = _prep(nf, wr_pad, br_pad, deg16, P)

    Xcat0, BDg0 = _gather_h[0](xn, xw, nf, BD)
    Xcat1, BDg1 = _gather_h[1](xn, xw, nf, BD)

    jp = jnp.arange(S * S)
    perm = (jp % S) * S + jp // S
    r = jnp.arange(S)
    Tm = (jp[None, :] // S == r[:, None]).astype(jnp.float32)
    Tm_pad = jnp.zeros((LN, S * S), jnp.float32).at[:S].set(Tm)
    F = (jnp.arange(LN)[:, None] % S == r[None, :]).astype(jnp.float32)

    dg2 = dg_list.reshape(E, 1)
    w1p = W_xi.T[:, perm]
    bxp = b_xi[perm].reshape(1, S * S)
    Hn0 = _dense(Xcat0, BDg0, dg2[:EH], w1p, bxp, Tm_pad, F)
    Hn1 = _dense(Xcat1, BDg1, dg2[EH:], w1p, bxp, Tm_pad, F)

    zvs = jnp.zeros((V * S,), jnp.float32)
    p0 = _scatter_h[0](xn, Hn0, zvs)
    p1 = _scatter_h[1](xn, Hn1, zvs)
    ns2 = _reduce_part2(p0, p1).reshape(V, S)

    out = _readout(nf, ns2, W_lin[:, :LN].T, W_lin[:, LN:].T,
                   b_lin.reshape(1, 3))
    return out
